# Initial kernel scaffold; baseline (speedup 1.0000x reference)
import jax
import jax.numpy as jnp
from jax.experimental import pallas as pl


def kernel(row, row_refs, row_embeddings):
    raise NotImplementedError("write your pallas kernel here")


# speedup vs baseline: 11.6666x; 11.6666x over previous
import functools

import jax
import jax.numpy as jnp
from jax import lax
from jax.experimental import pallas as pl
from jax.experimental.pallas import tpu as pltpu
from jax.experimental.pallas import tpu_sc as plsc

N = 512
D = 64
NSUB = 16
CHUNK = N // NSUB
LANES = 16


def _sc_body(rowflat_hbm, refs_hbm, emb_hbm, out_hbm,
             refs_v, flat_v, ids_v, rows_v, tmp_v, acc_sh, sem):
    cid = lax.axis_index("c")
    sid = lax.axis_index("s")
    base = sid * CHUNK

    pltpu.sync_copy(refs_hbm.at[pl.ds(base, CHUNK)], refs_v)

    for c in range(CHUNK // LANES):
        r16 = refs_v[pl.ds(c * LANES, LANES)]
        off = (base + c * LANES + lax.iota(jnp.int32, LANES)).astype(jnp.int32)
        flat_v[pl.ds(c * LANES, LANES)] = r16 * jnp.int32(N) + off

    pltpu.async_copy(rowflat_hbm.at[flat_v], ids_v, sem).wait()
    pltpu.async_copy(emb_hbm.at[ids_v], rows_v, sem).wait()

    for r in range(CHUNK):
        for c in range(D // LANES):
            tmp_v[r, pl.ds(c * LANES, LANES)] = jnp.zeros((LANES,), jnp.float32)
    pltpu.sync_copy(tmp_v, acc_sh.at[pl.ds(base, CHUNK)])
    plsc.subcore_barrier()

    pltpu.sync_copy(rows_v, acc_sh.at[refs_v], add=True)
    plsc.subcore_barrier()

    @pl.when(cid == 0)
    def _():
        pltpu.sync_copy(acc_sh.at[pl.ds(base, CHUNK)], tmp_v)
        pltpu.sync_copy(tmp_v, out_hbm.at[pl.ds(base, CHUNK)])


def kernel(row, row_refs, row_embeddings):
    mesh = plsc.VectorSubcoreMesh(core_axis_name="c", subcore_axis_name="s")
    k = functools.partial(
        pl.kernel,
        out_type=jax.ShapeDtypeStruct((N, D), jnp.float32),
        mesh=mesh,
        compiler_params=pltpu.CompilerParams(use_tc_tiling_on_sc=False),
        scratch_types=[
            pltpu.VMEM((CHUNK,), jnp.int32),
            pltpu.VMEM((CHUNK,), jnp.int32),
            pltpu.VMEM((CHUNK,), jnp.int32),
            pltpu.VMEM((CHUNK, D), jnp.float32),
            pltpu.VMEM((CHUNK, D), jnp.float32),
            pltpu.VMEM_SHARED((N, D), jnp.float32),
            pltpu.SemaphoreType.DMA,
        ],
    )(_sc_body)
    return k(row.reshape(-1), row_refs, row_embeddings)
